# Initial kernel scaffold; baseline (speedup 1.0000x reference)
import jax
import jax.numpy as jnp
from jax.experimental import pallas as pl


def kernel(node_info, branches, branch_info, params):
    raise NotImplementedError("write your pallas kernel here")


# speedup vs baseline: 23.4053x; 23.4053x over previous
import functools

import jax
import jax.numpy as jnp
from jax import lax
from jax.experimental import pallas as pl
from jax.experimental.pallas import tpu as pltpu
from jax.experimental.pallas import tpu_sc as plsc

_H = 32
_B = 4
_RN = 1000
_NSUB = 16
_NCORE = 2
_ECH = 256
_IDXW = 128


def _gelu(x):
    return 0.5 * x * (1.0 + lax.erf(x * 0.7071067811865476))


def _fold_ffn(params):
    out = []
    for p in params:
        s = p["gamma"] / jnp.sqrt(p["var"] + 1e-3)
        t = p["beta"] - p["mean"] * s
        w = p["W"] * s[:, None]
        b = (t @ p["W"] + p["b"]).reshape(1, -1)
        out.append((w, b))
    return out


def _l2norm(x):
    n = jnp.sqrt(jnp.maximum(jnp.sum(x * x, axis=-1, keepdims=True), 1e-12))
    return x / n


def _dot(a, b):
    return jnp.dot(a, b, preferred_element_type=jnp.float32,
                   precision=lax.Precision.HIGHEST)


def _stage_a_body(n_ref, wp0, bp0, wp1, bp1, wm0, bm0, wm1, bm1, x_ref, m_ref):
    for b in range(_B):
        nb = n_ref[b]
        h = _gelu(_dot(nb, wp0[...]) + bp0[...])
        x = _gelu(_dot(h, wp1[...]) + bp1[...])
        x_ref[:, b * _H:(b + 1) * _H] = x
        t = _gelu(_dot(x, wm0[...]) + bm0[...])
        m_ref[:, b * _H:(b + 1) * _H] = _gelu(_dot(t, wm1[...]) + bm1[...])


def _stage_b_body(x_ref, a0_ref, a1_ref, wna, wnb, bn0, wn1, bn1,
                  wm0, bm0, wm1, bm1, xn_ref, m_ref):
    for b in range(_B):
        sl = slice(b * _H, (b + 1) * _H)
        x = x_ref[:, sl]
        agg = a0_ref[:, sl] + a1_ref[:, sl]
        e = _gelu(_dot(x, wna[...]) + _dot(agg, wnb[...]) + bn0[...])
        e = _gelu(_dot(e, wn1[...]) + bn1[...])
        xn = _l2norm(e) + x
        xn_ref[:, sl] = xn
        t = _gelu(_dot(xn, wm0[...]) + bm0[...])
        m_ref[:, sl] = _gelu(_dot(t, wm1[...]) + bm1[...])


def _stage_c_body(x_ref, a0_ref, a1_ref, wna, wnb, bn0, wn1, bn1,
                  wq0, bq0, wq1, bq1, wout, bout, o_ref):
    cols = []
    for b in range(_B):
        sl = slice(b * _H, (b + 1) * _H)
        x = x_ref[:, sl]
        agg = a0_ref[:, sl] + a1_ref[:, sl]
        e = _gelu(_dot(x, wna[...]) + _dot(agg, wnb[...]) + bn0[...])
        e = _gelu(_dot(e, wn1[...]) + bn1[...])
        xf = _l2norm(e) + x
        p = _gelu(_dot(xf, wq0[...]) + bq0[...])
        p = _gelu(_dot(p, wq1[...]) + bq1[...])
        cols.append(jnp.maximum(_dot(p, wout[...]) + bout[...], 0.0))
    o_ref[...] = jnp.concatenate(cols, axis=1)


def _full_spec(arr):
    nd = arr.ndim
    return pl.BlockSpec(arr.shape, lambda i, _nd=nd: (0,) * _nd)


def _sc_conv_body(n_nodes, e_pad, m_hbm, src_hbm, dst_hbm, bw_hbm, out_hbm,
                  acc_sh, rows_v, src_v, dst_v, bw_v, sem):
    c = lax.axis_index("c")
    s = lax.axis_index("s")
    wid = s * _NCORE + c
    e_tile = e_pad // (_NCORE * _NSUB)
    n_chunks = e_tile // _ECH
    sub = _ECH // _IDXW

    def zero_body(i, carry):
        for k in range(8):
            rows_v[i, pl.ds(k * 16, 16)] = jnp.zeros((16,), jnp.float32)
        return carry
    lax.fori_loop(0, _ECH, zero_body, None)
    zr = (n_nodes // _NSUB) // 8 * 8
    zrem = n_nodes - _NSUB * zr
    zbase = s * zr
    off = 0
    while off < zr:
        seg = min(_ECH, zr - off)
        pltpu.sync_copy(rows_v.at[pl.ds(0, seg)],
                        acc_sh.at[pl.ds(zbase + off, seg)])
        off += seg
    if zrem:
        @pl.when(s == _NSUB - 1)
        def _zero_rem():
            pltpu.sync_copy(rows_v.at[pl.ds(0, zrem)],
                            acc_sh.at[pl.ds(_NSUB * zr, zrem)])
    plsc.subcore_barrier()

    for ch in range(n_chunks):
        goff = wid * e_tile + ch * _ECH
        grow = wid * (e_tile // _IDXW) + ch * sub
        pltpu.sync_copy(src_hbm.at[pl.ds(grow, sub)], src_v)
        pltpu.sync_copy(dst_hbm.at[pl.ds(grow, sub)], dst_v)
        for b in range(_B):
            pltpu.sync_copy(bw_hbm.at[pl.ds(b * e_pad + goff, _ECH)],
                            bw_v.at[b])

        for j in range(sub):
            pltpu.async_copy(m_hbm.at[src_v.at[j]],
                             rows_v.at[pl.ds(j * _IDXW, _IDXW)], sem).wait()

        def mul_body(g, carry):
            base = g * 16
            wv = [bw_v[b, pl.ds(base, 16)] for b in range(_B)]
            for j in range(16):
                e = base + j
                for b in range(_B):
                    w = jnp.full((16,), wv[b][j], jnp.float32)
                    lo = b * _H
                    rows_v[e, pl.ds(lo, 16)] = rows_v[e, pl.ds(lo, 16)] * w
                    rows_v[e, pl.ds(lo + 16, 16)] = rows_v[e, pl.ds(lo + 16, 16)] * w
            return carry
        lax.fori_loop(0, _ECH // 16, mul_body, None)

        for j in range(sub):
            pltpu.sync_copy(rows_v.at[pl.ds(j * _IDXW, _IDXW)],
                            acc_sh.at[dst_v.at[j]], add=True)

    plsc.subcore_barrier()
    obase = c * n_nodes
    off = 0
    while off < zr:
        seg = min(_ECH, zr - off)
        pltpu.sync_copy(acc_sh.at[pl.ds(zbase + off, seg)],
                        out_hbm.at[pl.ds(obase + zbase + off, seg)])
        off += seg
    if zrem:
        @pl.when(s == _NSUB - 1)
        def _write_rem():
            pltpu.sync_copy(acc_sh.at[pl.ds(_NSUB * zr, zrem)],
                            out_hbm.at[pl.ds(obase + _NSUB * zr, zrem)])


def _make_sc_conv(n_nodes, e_pad):
    mesh = plsc.VectorSubcoreMesh(core_axis_name="c", subcore_axis_name="s")
    body = functools.partial(_sc_conv_body, n_nodes, e_pad)
    return pl.kernel(
        body,
        out_type=jax.ShapeDtypeStruct((_NCORE * n_nodes, _B * _H), jnp.float32),
        mesh=mesh,
        scratch_types=[
            pltpu.VMEM_SHARED((n_nodes, _B * _H), jnp.float32),
            pltpu.VMEM((_ECH, _B * _H), jnp.float32),
            pltpu.VMEM((_ECH // _IDXW, _IDXW), jnp.int32),
            pltpu.VMEM((_ECH // _IDXW, _IDXW), jnp.int32),
            pltpu.VMEM((_B, _ECH), jnp.float32),
            pltpu.SemaphoreType.DMA,
        ],
        compiler_params=pltpu.CompilerParams(use_tc_tiling_on_sc=False),
    )


def kernel(node_info, branches, branch_info, params):
    bsz, n_nodes, d_feat = node_info.shape
    e_total = branches.shape[1]
    grain = _NCORE * _NSUB * _ECH
    e_pad = -(-e_total // grain) * grain

    idx = branches.astype(jnp.int32)
    idx = jnp.pad(idx, ((0, 0), (0, e_pad - e_total)))
    dst2 = idx[0].reshape(e_pad // _IDXW, _IDXW)
    src2 = idx[1].reshape(e_pad // _IDXW, _IDXW)
    bw = branch_info.reshape(bsz, e_total)
    bw = jnp.pad(bw, ((0, 0), (0, e_pad - e_total))).reshape(bsz * e_pad)

    pre = _fold_ffn(params["pre"])
    c1m = _fold_ffn(params["c1_msg"])
    c1n = _fold_ffn(params["c1_node"])
    c2m = _fold_ffn(params["c2_msg"])
    c2n = _fold_ffn(params["c2_node"])
    post = _fold_ffn(params["post"])

    grid = (n_nodes // _RN,)
    pk_spec = pl.BlockSpec((_RN, _B * _H), lambda i: (i, 0))
    agg_specs = [pl.BlockSpec((_RN, _B * _H), lambda i: (i, 0)),
                 pl.BlockSpec((_RN, _B * _H), lambda i, _o=n_nodes // _RN: (i + _o, 0))]
    pk_shape = jax.ShapeDtypeStruct((n_nodes, _B * _H), jnp.float32)

    sc_conv = _make_sc_conv(n_nodes, e_pad)

    wa = [pre[0][0], pre[0][1], pre[1][0], pre[1][1],
          c1m[0][0], c1m[0][1], c1m[1][0], c1m[1][1]]
    x_pk, m1_pk = pl.pallas_call(
        _stage_a_body,
        grid=grid,
        in_specs=[pl.BlockSpec((bsz, _RN, d_feat), lambda i: (0, i, 0))]
                 + [_full_spec(w) for w in wa],
        out_specs=[pk_spec, pk_spec],
        out_shape=[pk_shape, pk_shape],
    )(node_info, *wa)

    agg1 = sc_conv(m1_pk, src2, dst2, bw)

    wb = [c1n[0][0][:_H], c1n[0][0][_H:], c1n[0][1], c1n[1][0], c1n[1][1],
          c2m[0][0], c2m[0][1], c2m[1][0], c2m[1][1]]
    xn_pk, m2_pk = pl.pallas_call(
        _stage_b_body,
        grid=grid,
        in_specs=[pk_spec] + agg_specs + [_full_spec(w) for w in wb],
        out_specs=[pk_spec, pk_spec],
        out_shape=[pk_shape, pk_shape],
    )(x_pk, agg1, agg1, *wb)

    agg2 = sc_conv(m2_pk, src2, dst2, bw)

    wc = [c2n[0][0][:_H], c2n[0][0][_H:], c2n[0][1], c2n[1][0], c2n[1][1],
          post[0][0], post[0][1], post[1][0], post[1][1],
          params["Wout"], params["bout"].reshape(1, 1)]
    o_cols = pl.pallas_call(
        _stage_c_body,
        grid=grid,
        in_specs=[pk_spec] + agg_specs + [_full_spec(w) for w in wc],
        out_specs=pl.BlockSpec((_RN, _B), lambda i: (i, 0)),
        out_shape=jax.ShapeDtypeStruct((n_nodes, _B), jnp.float32),
    )(xn_pk, agg2, agg2, *wc)

    return o_cols.T.reshape(bsz, n_nodes, 1)
